# Initial kernel scaffold; baseline (speedup 1.0000x reference)
import jax
import jax.numpy as jnp
from jax.experimental import pallas as pl


def kernel(pos, z, batch, edge_index, emb, aw_W, aw_b, mlp1_W, mlp1_b, mlp2_W, mlp2_b, out1_W, out1_b, out2_W, out2_b, lin1_W, lin1_b, lin2_W, lin2_b):
    raise NotImplementedError("write your pallas kernel here")


# speedup vs baseline: 2.8640x; 2.8640x over previous
import functools

import jax
import jax.numpy as jnp
from jax import lax
from jax.experimental import pallas as pl
from jax.experimental.pallas import tpu as pltpu
from jax.experimental.pallas import tpu_sc as plsc

N = 10000
E = 320000
H = 128
G = 50
CUT = 10.0
NB = 6
NGRAPH = 64

NC = 2
NS = 16
NW = NC * NS

K = 128
NCH = E // K
NP = 10112
NCHN = NP // K

NPT = N // NS
ZR = 125

_LOG2 = 0.6931471805599453
_F32 = jnp.float32


def _ssp(x):
    return jax.nn.softplus(x) - _LOG2


def _nchunks(wid, total):
    return (total - 1 - wid) // NW + 1


_MESH = plsc.VectorSubcoreMesh(core_axis_name="c", subcore_axis_name="s")


D2R = 2560


@functools.partial(
    pl.kernel,
    out_type=(
        jax.ShapeDtypeStruct((D2R, 128), _F32),
        jax.ShapeDtypeStruct((NP, H), _F32),
    ),
    mesh=_MESH,
    scratch_types=[
        pltpu.VMEM((N,), _F32),
        pltpu.VMEM((N,), _F32),
        pltpu.VMEM((N,), _F32),
        pltpu.VMEM((K,), jnp.int32),
        pltpu.VMEM((K,), jnp.int32),
        pltpu.VMEM((8, K), _F32),
        pltpu.VMEM((K, H), _F32),
        pltpu.SemaphoreType.DMA,
    ],
    compiler_params=pltpu.CompilerParams(needs_layout_passes=False),
)
def _sc_prepass(posx_hbm, posy_hbm, posz_hbm, z_hbm, row_hbm, col_hbm,
                emb_hbm, d2_hbm, h0_hbm, px, py, pz, rowv, colv, dbuf, hbuf,
                sem):
    c = lax.axis_index("c")
    s = lax.axis_index("s")
    wid = s * NC + c

    pltpu.sync_copy(posx_hbm, px)
    pltpu.sync_copy(posy_hbm, py)
    pltpu.sync_copy(posz_hbm, pz)

    def ebody(j, carry):
        g = wid + j * NW
        for r in range(8):
            base = (g * 8 + r) * K
            pltpu.sync_copy(row_hbm.at[pl.ds(base, K)], rowv)
            pltpu.sync_copy(col_hbm.at[pl.ds(base, K)], colv)

            def sub(i, carry2):
                ri = rowv[pl.ds(i * 16, 16)]
                ci = colv[pl.ds(i * 16, 16)]
                dx = plsc.load_gather(px, [ri]) - plsc.load_gather(px, [ci])
                dy = plsc.load_gather(py, [ri]) - plsc.load_gather(py, [ci])
                dz = plsc.load_gather(pz, [ri]) - plsc.load_gather(pz, [ci])
                dbuf[r, pl.ds(i * 16, 16)] = dx * dx + dy * dy + dz * dz
                return carry2

            lax.fori_loop(0, K // 16, sub, 0)
        pltpu.sync_copy(dbuf, d2_hbm.at[pl.ds(g * 8, 8)])
        return carry

    lax.fori_loop(0, D2R // 8 // NW, ebody, 0)

    def nbody(j, carry):
        base = (wid + j * NW) * K
        pltpu.sync_copy(z_hbm.at[pl.ds(base, K)], rowv)
        pltpu.async_copy(emb_hbm.at[rowv], hbuf, sem).wait()
        pltpu.sync_copy(hbuf, h0_hbm.at[pl.ds(base, K)])
        return carry

    lax.fori_loop(0, _nchunks(wid, NCHN), nbody, 0)


@functools.partial(
    pl.kernel,
    out_type=jax.ShapeDtypeStruct((E, H), _F32),
    mesh=_MESH,
    scratch_types=[
        pltpu.VMEM((K,), jnp.int32),
        pltpu.VMEM((K, H), _F32),
        pltpu.SemaphoreType.DMA,
    ],
)
def _sc_gather(h1_hbm, row_hbm, hg_hbm, idxv, hbuf, sem):
    c = lax.axis_index("c")
    s = lax.axis_index("s")
    wid = s * NC + c

    def body(j, carry):
        base = (wid + j * NW) * K
        pltpu.sync_copy(row_hbm.at[pl.ds(base, K)], idxv)
        pltpu.async_copy(h1_hbm.at[idxv], hbuf, sem).wait()
        pltpu.sync_copy(hbuf, hg_hbm.at[pl.ds(base, K)])
        return carry

    lax.fori_loop(0, _nchunks(wid, NCH), body, 0)


NRS = 624
CH2 = 208


@functools.partial(
    pl.kernel,
    out_type=jax.ShapeDtypeStruct((2, N, H), _F32),
    mesh=_MESH,
    scratch_types=[
        pltpu.VMEM((K,), jnp.int32),
        pltpu.VMEM((K, H), _F32),
        pltpu.VMEM((CH2, H), _F32),
        pltpu.VMEM_SHARED((N, H), _F32),
        pltpu.SemaphoreType.DMA,
    ],
)
def _sc_scatter(msg_hbm, col_hbm, agg_hbm, colv, mbuf, zbuf, agg_sh, sem):
    c = lax.axis_index("c")
    s = lax.axis_index("s")
    wid = s * NC + c

    def zrow(r, carry):
        for j in range(H // 16):
            zbuf[r, pl.ds(j * 16, 16)] = jnp.zeros((16,), _F32)
        return carry

    lax.fori_loop(0, CH2, zrow, 0)
    for t in range(NRS // CH2):
        pltpu.sync_copy(zbuf, agg_sh.at[pl.ds(s * NRS + t * CH2, CH2)])

    @pl.when(s == 0)
    def _ztail():
        pltpu.sync_copy(zbuf.at[pl.ds(0, 16)], agg_sh.at[pl.ds(NS * NRS, 16)])

    plsc.subcore_barrier()

    def body(j, carry):
        base = (wid + j * NW) * K
        pltpu.sync_copy(col_hbm.at[pl.ds(base, K)], colv)
        pltpu.sync_copy(msg_hbm.at[pl.ds(base, K)], mbuf)
        pltpu.sync_copy(mbuf, agg_sh.at[colv], add=True)
        return carry

    lax.fori_loop(0, _nchunks(wid, NCH), body, 0)
    plsc.subcore_barrier()

    for t in range(NRS // CH2):
        r0 = s * NRS + t * CH2
        pltpu.sync_copy(agg_sh.at[pl.ds(r0, CH2)], zbuf)
        pltpu.sync_copy(zbuf, agg_hbm.at[c].at[pl.ds(r0, CH2)])

    @pl.when(s == 0)
    def _otail():
        pltpu.sync_copy(agg_sh.at[pl.ds(NS * NRS, 16)], zbuf.at[pl.ds(0, 16)])
        pltpu.sync_copy(zbuf.at[pl.ds(0, 16)],
                        agg_hbm.at[c].at[pl.ds(NS * NRS, 16)])


EP = D2R * 128
TEW = 4096


def _tc_edge_body(d2_ref, ea_ref):
    d2t = d2_ref[...].T
    delta = CUT / (G - 1)
    off = delta * lax.broadcasted_iota(jnp.int32, (1, 64), 1).astype(_F32)
    lane = lax.broadcasted_iota(jnp.int32, (1, 64), 1)
    for k in range(TEW // 128):
        ew = jnp.sqrt(d2t[:, k:k + 1] + 1e-12)
        ea = jnp.exp((-0.5 / (delta * delta)) * (ew - off) ** 2)
        cvals = 0.5 * (jnp.cos(ew * (jnp.pi / CUT)) + 1.0)
        ea_ref[pl.ds(k * 128, 128), :] = jnp.where(lane == 63, cvals, ea)


def _tc_edge(d2):
    return pl.pallas_call(
        _tc_edge_body,
        grid=(EP // TEW,),
        in_specs=[pl.BlockSpec((TEW // 128, 128), lambda i: (i, 0))],
        out_specs=pl.BlockSpec((TEW, 64), lambda i: (i, 0)),
        out_shape=jax.ShapeDtypeStruct((EP, 64), _F32),
    )(d2)


TEF = 3200


def _tc_filter_body(ea_ref, hg_ref, w1_ref, b1_ref, w2_ref, b2_ref,
                    msg_ref):
    ea = ea_ref[...]
    t = _ssp(jnp.dot(ea, w1_ref[...],
                     preferred_element_type=_F32) + b1_ref[...])
    wf = _ssp(jnp.dot(t, w2_ref[...],
                      preferred_element_type=_F32) + b2_ref[...])
    cc = ea[:, 63:64]
    msg_ref[...] = hg_ref[...] * (wf * cc)


def _tc_filter(ea, hg, w1, b1, w2, b2):
    return pl.pallas_call(
        _tc_filter_body,
        grid=(E // TEF,),
        in_specs=[
            pl.BlockSpec((TEF, 64), lambda i: (i, 0)),
            pl.BlockSpec((TEF, H), lambda i: (i, 0)),
            pl.BlockSpec((64, H), lambda i: (0, 0)),
            pl.BlockSpec((1, H), lambda i: (0, 0)),
            pl.BlockSpec((H, H), lambda i: (0, 0)),
            pl.BlockSpec((1, H), lambda i: (0, 0)),
        ],
        out_specs=pl.BlockSpec((TEF, H), lambda i: (i, 0)),
        out_shape=jax.ShapeDtypeStruct((E, H), _F32),
    )(ea, hg, w1, b1, w2, b2)


TN = 2000


def _tc_h1_body(h_ref, w_ref, b_ref, h1_ref):
    h1_ref[...] = jnp.dot(h_ref[...], w_ref[...],
                          preferred_element_type=_F32) + b_ref[...]


def _tc_h1(h, w, b):
    return pl.pallas_call(
        _tc_h1_body,
        grid=(N // TN,),
        in_specs=[
            pl.BlockSpec((TN, H), lambda i: (i, 0)),
            pl.BlockSpec((H, H), lambda i: (0, 0)),
            pl.BlockSpec((1, H), lambda i: (0, 0)),
        ],
        out_specs=pl.BlockSpec((TN, H), lambda i: (i, 0)),
        out_shape=jax.ShapeDtypeStruct((N, H), _F32),
    )(h, w, b)


def _tc_update_body(h_ref, a0_ref, a1_ref, o1w_ref, o1b_ref, o2w_ref, o2b_ref,
                    aww_ref, awb_ref, hn_ref, h1_ref):
    agg = a0_ref[...] + a1_ref[...]
    t = _ssp(jnp.dot(agg, o1w_ref[...],
                     preferred_element_type=_F32) + o1b_ref[...])
    hn = h_ref[...] + jnp.dot(t, o2w_ref[...],
                              preferred_element_type=_F32) + o2b_ref[...]
    hn_ref[...] = hn
    h1_ref[...] = jnp.dot(hn, aww_ref[...],
                          preferred_element_type=_F32) + awb_ref[...]


def _tc_update(h, a0, a1, o1w, o1b, o2w, o2b, aww, awb):
    wspec = pl.BlockSpec((H, H), lambda i: (0, 0))
    bspec = pl.BlockSpec((1, H), lambda i: (0, 0))
    nspec = pl.BlockSpec((TN, H), lambda i: (i, 0))
    return pl.pallas_call(
        _tc_update_body,
        grid=(N // TN,),
        in_specs=[nspec, nspec, nspec, wspec, bspec, wspec, bspec, wspec,
                  bspec],
        out_specs=[nspec, nspec],
        out_shape=[
            jax.ShapeDtypeStruct((N, H), _F32),
            jax.ShapeDtypeStruct((N, H), _F32),
        ],
    )(h, a0, a1, o1w, o1b, o2w, o2b, aww, awb)


TR = 400


def _tc_readout_body(h_ref, b_ref, l1w_ref, l1b_ref, l2w_ref, l2b_ref,
                     out_ref):
    i = pl.program_id(0)

    @pl.when(i == 0)
    def _init():
        out_ref[...] = jnp.zeros_like(out_ref)

    t = _ssp(jnp.dot(h_ref[...], l1w_ref[...],
                     preferred_element_type=_F32) + l1b_ref[...])
    hh = jnp.dot(t, l2w_ref[...], preferred_element_type=_F32) + l2b_ref[...]
    b = b_ref[0, 0, :]
    oh = (lax.broadcasted_iota(jnp.int32, (NGRAPH, 1), 0)
          == b[None, :]).astype(_F32)
    out_ref[...] += jnp.dot(oh, hh, preferred_element_type=_F32)


def _tc_readout(h, batch3, l1w, l1b, l2w, l2b):
    return pl.pallas_call(
        _tc_readout_body,
        grid=(N // TR,),
        in_specs=[
            pl.BlockSpec((TR, H), lambda i: (i, 0)),
            pl.BlockSpec((1, 1, TR), lambda i: (i, 0, 0)),
            pl.BlockSpec((H, 64), lambda i: (0, 0)),
            pl.BlockSpec((1, 64), lambda i: (0, 0)),
            pl.BlockSpec((64, 8), lambda i: (0, 0)),
            pl.BlockSpec((1, 8), lambda i: (0, 0)),
        ],
        out_specs=pl.BlockSpec((NGRAPH, 8), lambda i: (0, 0)),
        out_shape=jax.ShapeDtypeStruct((NGRAPH, 8), _F32),
    )(h, batch3, l1w, l1b, l2w, l2b)


def kernel(pos, z, batch, edge_index, emb, aw_W, aw_b, mlp1_W, mlp1_b,
           mlp2_W, mlp2_b, out1_W, out1_b, out2_W, out2_b, lin1_W, lin1_b,
           lin2_W, lin2_b):
    row = jnp.pad(edge_index[0].astype(jnp.int32), (0, EP - E))
    col = jnp.pad(edge_index[1].astype(jnp.int32), (0, EP - E))
    zp = jnp.pad(z.astype(jnp.int32), (0, NP - N))

    d2, h0p = _sc_prepass(pos[:, 0], pos[:, 1], pos[:, 2], zp, row, col, emb)
    ea = _tc_edge(d2)

    w1p = jnp.zeros((NB, 64, H), _F32).at[:, :G, :].set(mlp1_W)

    h = h0p[:N]
    h1 = _tc_h1(h, aw_W[0], aw_b[0].reshape(1, H))
    for b in range(NB):
        hg = _sc_gather(h1, row)
        msg = _tc_filter(ea, hg, w1p[b], mlp1_b[b].reshape(1, H),
                         mlp2_W[b], mlp2_b[b].reshape(1, H))
        agg = _sc_scatter(msg, col)
        bn = (b + 1) % NB
        h, h1 = _tc_update(h, agg[0], agg[1], out1_W[b],
                           out1_b[b].reshape(1, H), out2_W[b],
                           out2_b[b].reshape(1, H), aw_W[bn],
                           aw_b[bn].reshape(1, H))

    batch3 = batch.astype(jnp.int32).reshape(N // TR, 1, TR)
    l1w = lin1_W
    l1b = lin1_b.reshape(1, 64)
    l2w = jnp.zeros((64, 8), _F32).at[:, :1].set(lin2_W)
    l2b = jnp.zeros((1, 8), _F32).at[:, :1].set(lin2_b.reshape(1, 1))
    out = _tc_readout(h, batch3, l1w, l1b, l2w, l2b)
    return out[:, :1]
